# baseline (speedup 1.0000x reference)
import functools

import jax
import jax.numpy as jnp
from jax import lax
from jax.experimental import pallas as pl
from jax.experimental.pallas import tpu as pltpu
from jax.experimental.pallas import tpu_sc as plsc

B = 16384
F = 16
NU = 1000000
L = 16
NC = 2
NS = 16
NW = NC * NS
BPW = B // NW
CH = 128
NCHUNK = BPW // CH
SV = CH // L


def _sc_body(user_r, item_r, ufc, itc, uint_l, iint_l, g_r, out_r,
             uidx, iidx, uxb, ixb, ucols, icols, uintv, iintv, outv, gv, sem):
    c = lax.axis_index("c")
    s = lax.axis_index("s")
    wid = s * NC + c
    base = wid * BPW

    pltpu.sync_copy(user_r.at[pl.ds(base, BPW)], uidx)
    pltpu.sync_copy(item_r.at[pl.ds(base, BPW)], iidx)
    pltpu.sync_copy(g_r, gv)

    gvec = gv[...]

    def chunk_body(cc, carry):
        off = cc * CH
        for f in range(F):
            for sv in range(SV):
                sl = pl.ds(sv * L, L)
                src = pl.ds(off + sv * L, L)
                uxb[f, sl] = uidx[src] + f * NU
                ixb[f, sl] = iidx[src] + f * NU
        copies = []
        for f in range(F):
            copies.append(pltpu.async_copy(ufc.at[uxb.at[f]], ucols.at[f],
                                           sem))
            copies.append(pltpu.async_copy(itc.at[ixb.at[f]], icols.at[f],
                                           sem))
        copies.append(pltpu.async_copy(uint_l.at[uxb.at[0]], uintv, sem))
        copies.append(pltpu.async_copy(iint_l.at[ixb.at[0]], iintv, sem))
        for cp in copies:
            cp.wait()
        for sv in range(SV):
            sl = pl.ds(sv * L, L)
            acc = gvec + uintv[sl] + iintv[sl]
            for f in range(F):
                acc = acc + ucols[f, sl] * icols[f, sl]
            outv[pl.ds(off + sv * L, L)] = acc
        return carry

    lax.fori_loop(0, NCHUNK, chunk_body, 0)

    pltpu.sync_copy(outv, out_r.at[pl.ds(base, BPW)])


@functools.partial(
    pl.kernel,
    mesh=plsc.VectorSubcoreMesh(core_axis_name="c", subcore_axis_name="s"),
    out_type=jax.ShapeDtypeStruct((B,), jnp.float32),
    compiler_params=pltpu.CompilerParams(
        needs_layout_passes=False, use_tc_tiling_on_sc=False),
    scratch_types=[
        pltpu.VMEM((BPW,), jnp.int32),
        pltpu.VMEM((BPW,), jnp.int32),
        pltpu.VMEM((F, CH), jnp.int32),
        pltpu.VMEM((F, CH), jnp.int32),
        pltpu.VMEM((F, CH), jnp.float32),
        pltpu.VMEM((F, CH), jnp.float32),
        pltpu.VMEM((CH,), jnp.float32),
        pltpu.VMEM((CH,), jnp.float32),
        pltpu.VMEM((BPW,), jnp.float32),
        pltpu.VMEM((L,), jnp.float32),
        pltpu.SemaphoreType.DMA,
    ],
)
def _sc_kernel(*refs):
    _sc_body(*refs)


def kernel(user, item, user_factors, item_factors, user_intercepts,
           item_intercepts, global_intercept):
    ufc = user_factors.T.reshape(-1)
    itc = item_factors.T.reshape(-1)
    uint_l = user_intercepts.reshape(-1)
    iint_l = item_intercepts.reshape(-1)
    g_r = jnp.broadcast_to(global_intercept.reshape(()), (L,))
    return _sc_kernel(user, item, ufc, itc, uint_l, iint_l, g_r)


# speedup vs baseline: 3.3009x; 3.3009x over previous
import functools

import jax
import jax.numpy as jnp
from jax import lax
from jax.experimental import pallas as pl
from jax.experimental.pallas import tpu as pltpu
from jax.experimental.pallas import tpu_sc as plsc

B = 16384
F = 16
L = 16
NC = 2
NS = 16
NW = NC * NS
BPW = B // NW
CH = 128
NCHUNK = BPW // CH


def _sc_body(user_r, item_r, uf, itf, uint_r, iint_r, g_r, out_r,
             uidx, iidx, urows, irows, uintv, iintv, outv, gv, sem):
    c = lax.axis_index("c")
    s = lax.axis_index("s")
    wid = s * NC + c
    base = wid * BPW

    pltpu.sync_copy(user_r.at[wid], uidx)
    pltpu.sync_copy(item_r.at[wid], iidx)
    pltpu.sync_copy(g_r, gv)

    copies = []
    for ci in range(NCHUNK):
        sl = pl.ds(ci * CH, CH)
        copies.append(pltpu.async_copy(uf.at[uidx.at[ci]], urows.at[sl], sem))
        copies.append(pltpu.async_copy(itf.at[iidx.at[ci]], irows.at[sl], sem))
        copies.append(pltpu.async_copy(uint_r.at[uidx.at[ci]], uintv.at[sl], sem))
        copies.append(pltpu.async_copy(iint_r.at[iidx.at[ci]], iintv.at[sl], sem))
    for cp in copies:
        cp.wait()

    iota = lax.iota(jnp.int32, L)
    gvec = gv[...]

    def tile_body(t, carry):
        r0 = t * L
        acc = uintv[pl.ds(r0, L)] + iintv[pl.ds(r0, L)] + gvec
        for j in range(L):
            p = urows[r0 + j, :] * irows[r0 + j, :]
            s = jnp.sum(p)
            acc = jnp.where(iota == j, acc + s, acc)
        outv[pl.ds(r0, L)] = acc
        return carry

    lax.fori_loop(0, BPW // L, tile_body, 0)

    pltpu.sync_copy(outv, out_r.at[pl.ds(base, BPW)])


@functools.partial(
    pl.kernel,
    mesh=plsc.VectorSubcoreMesh(core_axis_name="c", subcore_axis_name="s"),
    out_type=jax.ShapeDtypeStruct((B,), jnp.float32),
    compiler_params=pltpu.CompilerParams(
        needs_layout_passes=False, use_tc_tiling_on_sc=False),
    scratch_types=[
        pltpu.VMEM((NCHUNK, CH), jnp.int32),
        pltpu.VMEM((NCHUNK, CH), jnp.int32),
        pltpu.VMEM((BPW, F), jnp.float32),
        pltpu.VMEM((BPW, F), jnp.float32),
        pltpu.VMEM((BPW,), jnp.float32),
        pltpu.VMEM((BPW,), jnp.float32),
        pltpu.VMEM((BPW,), jnp.float32),
        pltpu.VMEM((L,), jnp.float32),
        pltpu.SemaphoreType.DMA,
    ],
)
def _sc_kernel(*refs):
    _sc_body(*refs)


def kernel(user, item, user_factors, item_factors, user_intercepts,
           item_intercepts, global_intercept):
    user_r = user.reshape(NW, NCHUNK, CH)
    item_r = item.reshape(NW, NCHUNK, CH)
    one = (global_intercept * 0.0 + 1.0).reshape(())
    uint_r = user_intercepts.reshape(-1) * one
    iint_r = item_intercepts.reshape(-1) * one
    g_r = jnp.broadcast_to(global_intercept.reshape(()), (L,))
    return _sc_kernel(user_r, item_r, user_factors, item_factors,
                      uint_r, iint_r, g_r)
